# baseline (speedup 1.0000x reference)
import functools
import math

import jax
import jax.numpy as jnp
from jax import lax
from jax.experimental import pallas as pl
from jax.experimental.pallas import tpu as pltpu
from jax.experimental.pallas import tpu_sc as plsc

_NA = 100
_NF = 512
_NW = 32
_FPT = _NF // _NW
_FSTR = _NA * 3
_NB = 180
_NBP = 192
_NBLK = 7
_WIN = 8

_ACOS = (1.5707963050, -0.2145988016, 0.0889789874, -0.0501743046,
         0.0308918810, -0.0170881256, 0.0066700901, -0.0012624911)
_PI = math.pi
_USCALE = 179.0 / math.pi
_EJ = tuple(math.exp(-(j + 0.5)) for j in range(_WIN))


def _rsqrt_nr(a, iters=3):
    bits = lax.bitcast_convert_type(a, jnp.int32)
    y = lax.bitcast_convert_type(jnp.int32(0x5F3759DF) - (bits >> 1),
                                 jnp.float32)
    for _ in range(iters):
        y = y * (1.5 - 0.5 * a * y * y)
    return y


def _acos_poly(t):
    p = jnp.full((16,), _ACOS[7], jnp.float32)
    for c in (_ACOS[6], _ACOS[5], _ACOS[4], _ACOS[3],
              _ACOS[2], _ACOS[1], _ACOS[0]):
        p = p * t + c
    omt = 1.0 - t
    return omt * _rsqrt_nr(omt, 2) * p


def _angle_u(dotv, n1, n2, valid):
    cosv = dotv * _rsqrt_nr(n1 * n2, 3)
    cosv = jnp.clip(cosv, -1.0 + 1e-7, 1.0 - 1e-7)
    cosv = jnp.where(valid, cosv, 0.0)
    th = _acos_poly(jnp.abs(cosv))
    theta = jnp.where(cosv < 0.0, _PI - th, th)
    return theta * _USCALE


def _sc_body(x_hbm, out_hbm, xyz_v, hist_v, loc_v):
    wid = lax.axis_index("c") * 16 + lax.axis_index("s")
    pltpu.sync_copy(x_hbm.at[wid], xyz_v)
    lanes = lax.iota(jnp.int32, 16)
    lanes3 = lanes * 3
    lane_base = lanes * _NBP
    zero16 = jnp.zeros((16,), jnp.float32)
    for lane in range(16):
        for b in range(_NBP // 16):
            hist_v[pl.ds(lane * _NBP + b * 16, 16)] = zero16

    def frame_body(ff, carry):
        foff = ff * _FSTR
        for cb in range(_NBLK):
            base = 1 + cb * 16
            gl = foff + (base - 1) * 3 + lanes3
            dotv = zero16
            n1 = zero16
            n2 = zero16
            ws1 = zero16
            ws2 = zero16
            for d in range(3):
                left = plsc.load_gather(xyz_v, [gl + d])
                ctr = plsc.load_gather(xyz_v, [gl + (d + 3)])
                right = plsc.load_gather(xyz_v, [gl + (d + 6)])
                v1 = left - ctr
                v2 = right - ctr
                dotv = dotv + v1 * v2
                n1 = n1 + v1 * v1
                n2 = n2 + v2 * v2
                w1 = v1 + (jnp.where(v1 >= 10.0, -20.0, 0.0)
                           + jnp.where(v1 < -10.0, 20.0, 0.0))
                w2 = v2 + (jnp.where(v2 >= 10.0, -20.0, 0.0)
                           + jnp.where(v2 < -10.0, 20.0, 0.0))
                ws1 = ws1 + w1 * w1
                ws2 = ws2 + w2 * w2
            valid = ((ws1 < 9.0) & (ws1 != 0.0)
                     & (ws2 < 9.0) & (ws2 != 0.0))
            if cb == _NBLK - 1:
                valid = valid & (lanes < (_NA - 2) - (base - 1))
            u = _angle_u(dotv, n1, n2, valid)
            s_i = jnp.clip(u.astype(jnp.int32) - 3, 0, _NB - _WIN)
            d0 = u - s_i.astype(jnp.float32)
            w = jnp.exp(-0.5 * d0 * d0)
            g = jnp.exp(d0)
            idx = lane_base + s_i
            for j in range(_WIN):
                plsc.addupdate_scatter(hist_v, [idx + j], w, mask=valid)
                if j + 1 < _WIN:
                    w = w * (g * _EJ[j])
        return carry

    lax.fori_loop(0, _FPT, frame_body, 0)

    for b in range(_NBP // 16):
        acc = hist_v[pl.ds(b * 16, 16)]
        for lane in range(1, 16):
            acc = acc + hist_v[pl.ds(lane * _NBP + b * 16, 16)]
        loc_v[pl.ds(b * 16, 16)] = acc
    pltpu.sync_copy(loc_v, out_hbm.at[wid])

    @pl.when(wid == 0)
    def _fb():
        a0 = plsc.load_gather(xyz_v, [lanes])
        a1 = plsc.load_gather(xyz_v, [lanes + 3])
        a2 = plsc.load_gather(xyz_v, [lanes + 6])
        fm = lanes < 3
        v1 = jnp.where(fm, a1 - a0, 0.0)
        v2 = jnp.where(fm, a2 - a0, 0.0)
        dotv = jnp.full((16,), jnp.sum(v1 * v2), jnp.float32)
        n1 = jnp.full((16,), jnp.sum(v1 * v1), jnp.float32)
        n2 = jnp.full((16,), jnp.sum(v2 * v2), jnp.float32)
        tru = jnp.ones((16,), jnp.bool_)
        u = _angle_u(dotv, n1, n2, tru)
        for b in range(_NBP // 16):
            dd = u - (b * 16 + lanes).astype(jnp.float32)
            w = jnp.exp(-0.5 * dd * dd)
            if (b + 1) * 16 > _NB:
                w = jnp.where(b * 16 + lanes < _NB, w, 0.0)
            loc_v[pl.ds(b * 16, 16)] = w
        pltpu.sync_copy(loc_v, out_hbm.at[_NW])


_sc_hist = pl.kernel(
    _sc_body,
    out_type=jax.ShapeDtypeStruct((40, _NBP), jnp.float32),
    mesh=plsc.VectorSubcoreMesh(core_axis_name="c", subcore_axis_name="s",
                                num_cores=2, num_subcores=16),
    compiler_params=pltpu.CompilerParams(needs_layout_passes=False),
    scratch_types=[
        pltpu.VMEM((_FPT * _FSTR,), jnp.float32),
        pltpu.VMEM((16 * _NBP,), jnp.float32),
        pltpu.VMEM((_NBP,), jnp.float32),
    ],
)


def _ep_body(parts_ref, out_ref):
    p = parts_ref[...]
    hist = p[:_NW].sum(axis=0)
    fb = p[_NW]
    sel = jnp.where(hist.sum() > 0.0, hist, fb)
    out_ref[...] = (sel / sel.sum())[:_NB]


_epilogue = pl.pallas_call(
    _ep_body,
    out_shape=jax.ShapeDtypeStruct((_NB,), jnp.float32),
)


def kernel(xyz):
    xyz = xyz.reshape(-1, _NA, 3)
    x = xyz.reshape(_NW, _FPT * _FSTR)
    parts = _sc_hist(x)
    return _epilogue(parts)


# speedup vs baseline: 2.0660x; 2.0660x over previous
import functools
import math

import jax
import jax.numpy as jnp
from jax import lax
from jax.experimental import pallas as pl
from jax.experimental.pallas import tpu as pltpu
from jax.experimental.pallas import tpu_sc as plsc

_NA = 100
_NF = 512
_NW = 32
_FPT = _NF // _NW
_PADA = 120
_NB = 180
_NBP = 192
_NBLK = 7
_WIN = 8

_ACOS = (1.5707963050, -0.2145988016, 0.0889789874, -0.0501743046,
         0.0308918810, -0.0170881256, 0.0066700901, -0.0012624911)
_PI = math.pi
_USCALE = 179.0 / math.pi


def _rsqrt_nr(a, iters=3):
    bits = lax.bitcast_convert_type(a, jnp.int32)
    y = lax.bitcast_convert_type(jnp.int32(0x5F3759DF) - (bits >> 1),
                                 jnp.float32)
    for _ in range(iters):
        y = y * (1.5 - 0.5 * a * y * y)
    return y


def _acos_poly(t):
    p = jnp.full((16,), _ACOS[7], jnp.float32)
    for c in (_ACOS[6], _ACOS[5], _ACOS[4], _ACOS[3],
              _ACOS[2], _ACOS[1], _ACOS[0]):
        p = p * t + c
    omt = 1.0 - t
    return omt * _rsqrt_nr(omt, 2) * p


def _angle_u(dotv, n1, n2, valid):
    cosv = dotv * _rsqrt_nr(n1 * n2, 3)
    cosv = jnp.clip(cosv, -1.0 + 1e-7, 1.0 - 1e-7)
    cosv = jnp.where(valid, cosv, 0.0)
    th = _acos_poly(jnp.abs(cosv))
    theta = jnp.where(cosv < 0.0, _PI - th, th)
    return theta * _USCALE


def _sc_body(x_hbm, out_hbm, xyz_v, hist_v, loc_v):
    wid = lax.axis_index("c") * 16 + lax.axis_index("s")
    pltpu.sync_copy(x_hbm.at[wid], xyz_v)
    lanes = lax.iota(jnp.int32, 16)
    lane_base = lanes * _NBP
    zero16 = jnp.zeros((16,), jnp.float32)
    for lane in range(16):
        for b in range(_NBP // 16):
            hist_v[pl.ds(lane * _NBP + b * 16, 16)] = zero16

    def frame_body(ff, carry):
        for cb in range(_NBLK):
            base = 1 + cb * 16
            dotv = zero16
            n1 = zero16
            n2 = zero16
            ws1 = zero16
            ws2 = zero16
            for d in range(3):
                left = xyz_v[d, ff, pl.ds(base - 1, 16)]
                ctr = xyz_v[d, ff, pl.ds(base, 16)]
                right = xyz_v[d, ff, pl.ds(base + 1, 16)]
                v1 = left - ctr
                v2 = right - ctr
                dotv = dotv + v1 * v2
                n1 = n1 + v1 * v1
                n2 = n2 + v2 * v2
                w1 = v1 + (jnp.where(v1 >= 10.0, -20.0, 0.0)
                           + jnp.where(v1 < -10.0, 20.0, 0.0))
                w2 = v2 + (jnp.where(v2 >= 10.0, -20.0, 0.0)
                           + jnp.where(v2 < -10.0, 20.0, 0.0))
                ws1 = ws1 + w1 * w1
                ws2 = ws2 + w2 * w2
            valid = ((ws1 < 9.0) & (ws1 != 0.0)
                     & (ws2 < 9.0) & (ws2 != 0.0))
            if cb == _NBLK - 1:
                valid = valid & (lanes < (_NA - 2) - (base - 1))
            u = _angle_u(dotv, n1, n2, valid)
            s_i = jnp.clip(u.astype(jnp.int32) - 3, 0, _NB - _WIN)
            d0 = u - s_i.astype(jnp.float32)
            idx = lane_base + s_i
            for j in range(_WIN):
                dd = d0 - float(j)
                w = jnp.exp(-0.5 * dd * dd)
                plsc.addupdate_scatter(hist_v, [idx + j], w, mask=valid)
        return carry

    lax.fori_loop(0, _FPT, frame_body, 0)

    for b in range(_NBP // 16):
        acc = hist_v[pl.ds(b * 16, 16)]
        for lane in range(1, 16):
            acc = acc + hist_v[pl.ds(lane * _NBP + b * 16, 16)]
        loc_v[pl.ds(b * 16, 16)] = acc
    pltpu.sync_copy(loc_v, out_hbm.at[wid])

    @pl.when(wid == 0)
    def _fb():
        pp = zero16
        q1 = zero16
        q2 = zero16
        for d in range(3):
            r0 = xyz_v[d, 0, pl.ds(0, 16)]
            r1 = xyz_v[d, 0, pl.ds(1, 16)]
            r2 = xyz_v[d, 0, pl.ds(2, 16)]
            w1 = r1 - r0
            w2 = r2 - r0
            pp = pp + w1 * w2
            q1 = q1 + w1 * w1
            q2 = q2 + w2 * w2
        m0 = lanes == 0
        dotv = jnp.full((16,), jnp.sum(jnp.where(m0, pp, 0.0)),
                        jnp.float32)
        n1 = jnp.full((16,), jnp.sum(jnp.where(m0, q1, 0.0)), jnp.float32)
        n2 = jnp.full((16,), jnp.sum(jnp.where(m0, q2, 0.0)), jnp.float32)
        tru = jnp.ones((16,), jnp.bool_)
        u = _angle_u(dotv, n1, n2, tru)
        for b in range(_NBP // 16):
            dd = u - (b * 16 + lanes).astype(jnp.float32)
            w = jnp.exp(-0.5 * dd * dd)
            if (b + 1) * 16 > _NB:
                w = jnp.where(b * 16 + lanes < _NB, w, 0.0)
            loc_v[pl.ds(b * 16, 16)] = w
        pltpu.sync_copy(loc_v, out_hbm.at[_NW])


_sc_hist = pl.kernel(
    _sc_body,
    out_type=jax.ShapeDtypeStruct((40, _NBP), jnp.float32),
    mesh=plsc.VectorSubcoreMesh(core_axis_name="c", subcore_axis_name="s",
                                num_cores=2, num_subcores=16),
    compiler_params=pltpu.CompilerParams(needs_layout_passes=False),
    scratch_types=[
        pltpu.VMEM((3, _FPT, _PADA), jnp.float32),
        pltpu.VMEM((16 * _NBP,), jnp.float32),
        pltpu.VMEM((_NBP,), jnp.float32),
    ],
)


def _ep_body(parts_ref, out_ref):
    p = parts_ref[...]
    hist = p[:_NW].sum(axis=0)
    fb = p[_NW]
    sel = jnp.where(hist.sum() > 0.0, hist, fb)
    out_ref[...] = (sel / sel.sum())[:_NB]


_epilogue = pl.pallas_call(
    _ep_body,
    out_shape=jax.ShapeDtypeStruct((_NB,), jnp.float32),
)


def kernel(xyz):
    xyz = xyz.reshape(-1, _NA, 3)
    x = jnp.transpose(xyz, (2, 0, 1))
    x = jnp.pad(x, ((0, 0), (0, 0), (0, _PADA - _NA)))
    x = x.reshape(3, _NW, _FPT, _PADA).transpose(1, 0, 2, 3)
    parts = _sc_hist(x)
    return _epilogue(parts)
